# baseline (speedup 1.0000x reference)
import functools

import jax
import jax.numpy as jnp
from jax import lax
from jax.experimental import pallas as pl
from jax.experimental.pallas import tpu as pltpu
from jax.experimental.pallas import tpu_sc as plsc


def _make_gather(B: int, D: int, NC: int, NS: int, C: int):
    NW = NC * NS
    b_per_w = B // NW
    n_chunks = b_per_w // C
    mesh = plsc.VectorSubcoreMesh(core_axis_name="c", subcore_axis_name="s")

    @functools.partial(
        pl.kernel,
        mesh=mesh,
        out_type=jax.ShapeDtypeStruct((B, D), jnp.float32),
        scratch_types=[
            pltpu.VMEM((C,), jnp.int32),
            pltpu.VMEM((C, D), jnp.float32),
            pltpu.SemaphoreType.DMA,
        ],
    )
    def gather_kernel(table_hbm, idx_hbm, out_hbm, idx_v, rows_v, sem):
        wid = lax.axis_index("s") * NC + lax.axis_index("c")
        base = wid * b_per_w

        def body(g, carry):
            off = base + g * C
            pltpu.sync_copy(idx_hbm.at[pl.ds(off, C)], idx_v)
            pltpu.async_copy(table_hbm.at[idx_v], rows_v, sem).wait()
            pltpu.sync_copy(rows_v, out_hbm.at[pl.ds(off, C)])
            return carry

        lax.fori_loop(0, n_chunks, body, 0)

    return gather_kernel


def kernel(x, table):
    B = x.shape[0] * x.shape[1]
    D = table.shape[1]
    info = plsc.get_sparse_core_info()
    NC, NS = info.num_cores, info.num_subcores
    gather = _make_gather(B, D, NC, NS, C=64)
    out = gather(table, x.reshape(B).astype(jnp.int32))
    return out.reshape(x.shape[0], x.shape[1], D)


# speedup vs baseline: 1.0030x; 1.0030x over previous
import functools

import jax
import jax.numpy as jnp
from jax import lax
from jax.experimental import pallas as pl
from jax.experimental.pallas import tpu as pltpu
from jax.experimental.pallas import tpu_sc as plsc


def _make_gather(B: int, D: int, NC: int, NS: int, C: int):
    NW = NC * NS
    b_per_w = B // NW
    n_chunks = b_per_w // C
    assert n_chunks % 2 == 0 and n_chunks >= 4
    mesh = plsc.VectorSubcoreMesh(core_axis_name="c", subcore_axis_name="s")

    @functools.partial(
        pl.kernel,
        mesh=mesh,
        out_type=jax.ShapeDtypeStruct((B, D), jnp.float32),
        scratch_types=[
            pltpu.VMEM((b_per_w,), jnp.int32),
            pltpu.VMEM((C, D), jnp.float32),
            pltpu.VMEM((C, D), jnp.float32),
            pltpu.SemaphoreType.DMA,
            pltpu.SemaphoreType.DMA,
            pltpu.SemaphoreType.DMA,
            pltpu.SemaphoreType.DMA,
        ],
    )
    def gather_kernel(table_hbm, idx_hbm, out_hbm, idx_v, rows0, rows1,
                      semg0, semg1, semo0, semo1):
        wid = lax.axis_index("s") * NC + lax.axis_index("c")
        base = wid * b_per_w
        rows = (rows0, rows1)
        semg = (semg0, semg1)
        semo = (semo0, semo1)

        pltpu.sync_copy(idx_hbm.at[pl.ds(base, b_per_w)], idx_v)

        def start_gather(g, b):
            pltpu.async_copy(
                table_hbm.at[idx_v.at[pl.ds(g * C, C)]], rows[b], semg[b])

        def start_out(g, b):
            pltpu.async_copy(rows[b], out_hbm.at[pl.ds(base + g * C, C)],
                             semo[b])

        def wait(sem, buf, g):
            pltpu.make_async_copy(buf, out_hbm.at[pl.ds(base + g * C, C)],
                                  sem).wait()

        start_gather(0, 0)
        start_gather(1, 1)
        pltpu.make_async_copy(table_hbm.at[idx_v.at[pl.ds(0, C)]], rows0,
                              semg0).wait()
        start_out(0, 0)
        pltpu.make_async_copy(table_hbm.at[idx_v.at[pl.ds(C, C)]], rows1,
                              semg1).wait()
        start_out(1, 1)

        def body(t, carry):
            for b in range(2):
                g = 2 * t + b
                wait(semo[b], rows[b], g)
                start_gather(g, b)
                pltpu.make_async_copy(
                    table_hbm.at[idx_v.at[pl.ds(g * C, C)]], rows[b],
                    semg[b]).wait()
                start_out(g, b)
            return carry

        lax.fori_loop(1, n_chunks // 2, body, 0)

        wait(semo0, rows0, n_chunks - 2)
        wait(semo1, rows1, n_chunks - 1)

    return gather_kernel


def kernel(x, table):
    B = x.shape[0] * x.shape[1]
    D = table.shape[1]
    info = plsc.get_sparse_core_info()
    NC, NS = info.num_cores, info.num_subcores
    gather = _make_gather(B, D, NC, NS, C=64)
    out = gather(table, x.reshape(B).astype(jnp.int32))
    return out.reshape(x.shape[0], x.shape[1], D)


# speedup vs baseline: 1.0050x; 1.0020x over previous
import functools

import jax
import jax.numpy as jnp
from jax import lax
from jax.experimental import pallas as pl
from jax.experimental.pallas import tpu as pltpu
from jax.experimental.pallas import tpu_sc as plsc

_NBUF = 4


def _make_gather(B: int, D: int, NC: int, NS: int, C: int):
    NW = NC * NS
    b_per_w = B // NW
    n_chunks = b_per_w // C
    assert n_chunks % _NBUF == 0 and n_chunks >= 2 * _NBUF
    mesh = plsc.VectorSubcoreMesh(core_axis_name="c", subcore_axis_name="s")

    @functools.partial(
        pl.kernel,
        mesh=mesh,
        out_type=jax.ShapeDtypeStruct((B, D), jnp.float32),
        scratch_types=[
            pltpu.VMEM((b_per_w,), jnp.int32),
        ] + [pltpu.VMEM((C, D), jnp.float32) for _ in range(_NBUF)]
          + [pltpu.SemaphoreType.DMA for _ in range(2 * _NBUF)],
    )
    def gather_kernel(table_hbm, idx_hbm, out_hbm, idx_v, *bufs_sems):
        rows = bufs_sems[:_NBUF]
        semg = bufs_sems[_NBUF:2 * _NBUF]
        semo = bufs_sems[2 * _NBUF:]
        wid = lax.axis_index("s") * NC + lax.axis_index("c")
        base = wid * b_per_w

        pltpu.sync_copy(idx_hbm.at[pl.ds(base, b_per_w)], idx_v)

        def start_gather(g, b):
            pltpu.async_copy(
                table_hbm.at[idx_v.at[pl.ds(g * C, C)]], rows[b], semg[b])

        def wait_gather(g, b):
            pltpu.make_async_copy(
                table_hbm.at[idx_v.at[pl.ds(g * C, C)]], rows[b],
                semg[b]).wait()

        def start_out(g, b):
            pltpu.async_copy(rows[b], out_hbm.at[pl.ds(base + g * C, C)],
                             semo[b])

        def wait_out(g, b):
            pltpu.make_async_copy(rows[b],
                                  out_hbm.at[pl.ds(base + g * C, C)],
                                  semo[b]).wait()

        for b in range(_NBUF):
            start_gather(b, b)
        for b in range(_NBUF):
            wait_gather(b, b)
            start_out(b, b)

        def body(t, carry):
            for b in range(_NBUF):
                g = _NBUF * t + b
                wait_out(g, b)
                start_gather(g, b)
            for b in range(_NBUF):
                g = _NBUF * t + b
                wait_gather(g, b)
                start_out(g, b)
            return carry

        lax.fori_loop(1, n_chunks // _NBUF, body, 0)

        for b in range(_NBUF):
            wait_out(n_chunks - _NBUF + b, b)

    return gather_kernel


def kernel(x, table):
    B = x.shape[0] * x.shape[1]
    D = table.shape[1]
    info = plsc.get_sparse_core_info()
    NC, NS = info.num_cores, info.num_subcores
    gather = _make_gather(B, D, NC, NS, C=32)
    out = gather(table, x.reshape(B).astype(jnp.int32))
    return out.reshape(x.shape[0], x.shape[1], D)


# speedup vs baseline: 1.3084x; 1.3019x over previous
import functools

import jax
import jax.numpy as jnp
from jax import lax
from jax.experimental import pallas as pl
from jax.experimental.pallas import tpu as pltpu
from jax.experimental.pallas import tpu_sc as plsc

_NBUF = 2


def _make_lookup(B: int, D: int, NC: int, NS: int, C: int):
    NW = NC * NS
    b_per_w = B // NW
    n_chunks = b_per_w // C
    n_vecs = D // 16
    assert n_chunks % _NBUF == 0 and n_chunks >= 2 * _NBUF
    mesh = plsc.VectorSubcoreMesh(core_axis_name="c", subcore_axis_name="s")

    @functools.partial(
        pl.kernel,
        mesh=mesh,
        out_type=jax.ShapeDtypeStruct((B, D), jnp.float32),
        scratch_types=[
            pltpu.VMEM((b_per_w,), jnp.int32),
            pltpu.VMEM((16, D), jnp.float32),
        ] + [pltpu.VMEM((C, D), jnp.float32) for _ in range(_NBUF)]
          + [pltpu.SemaphoreType.DMA for _ in range(_NBUF)],
    )
    def lookup_kernel(table_hbm, idx_hbm, out_hbm, idx_v, table_v,
                      *bufs_sems):
        rows = bufs_sems[:_NBUF]
        semo = bufs_sems[_NBUF:]
        wid = lax.axis_index("s") * NC + lax.axis_index("c")
        base = wid * b_per_w

        pltpu.sync_copy(table_hbm, table_v)
        pltpu.sync_copy(idx_hbm.at[pl.ds(base, b_per_w)], idx_v)

        def fill(g, b):
            buf = rows[b]

            def blk(jv, carry):
                vec = idx_v[pl.ds(g * C + jv * 16, 16)]
                for l in range(16):
                    r = vec[l]
                    j = jv * 16 + l
                    for d in range(n_vecs):
                        buf[j, pl.ds(16 * d, 16)] = (
                            table_v[r, pl.ds(16 * d, 16)])
                return carry

            lax.fori_loop(0, C // 16, blk, 0)

        def start_out(g, b):
            pltpu.async_copy(rows[b], out_hbm.at[pl.ds(base + g * C, C)],
                             semo[b])

        def wait_out(g, b):
            pltpu.make_async_copy(rows[b],
                                  out_hbm.at[pl.ds(base + g * C, C)],
                                  semo[b]).wait()

        def body(t, carry):
            for b in range(_NBUF):
                g = _NBUF * t + b

                @pl.when(g >= _NBUF)
                def _():
                    wait_out(g, b)

                fill(g, b)
                start_out(g, b)
            return carry

        lax.fori_loop(0, n_chunks // _NBUF, body, 0)

        for b in range(_NBUF):
            wait_out(n_chunks - _NBUF + b, b)

    return lookup_kernel


def kernel(x, table):
    B = x.shape[0] * x.shape[1]
    D = table.shape[1]
    info = plsc.get_sparse_core_info()
    NC, NS = info.num_cores, info.num_subcores
    lookup = _make_lookup(B, D, NC, NS, C=64)
    out = lookup(table, x.reshape(B).astype(jnp.int32))
    return out.reshape(x.shape[0], x.shape[1], D)


# speedup vs baseline: 1.6354x; 1.2500x over previous
import functools

import jax
import jax.numpy as jnp
from jax import lax
from jax.experimental import pallas as pl
from jax.experimental.pallas import tpu as pltpu
from jax.experimental.pallas import tpu_sc as plsc

_NBUF = 2


def _make_lookup(B: int, D: int, NC: int, NS: int, C: int):
    NW = NC * NS
    b_per_w = B // NW
    n_chunks = b_per_w // C
    n_vecs = D // 16
    assert n_chunks % _NBUF == 0 and n_chunks >= 2 * _NBUF
    mesh = plsc.VectorSubcoreMesh(core_axis_name="c", subcore_axis_name="s")

    @functools.partial(
        pl.kernel,
        mesh=mesh,
        out_type=jax.ShapeDtypeStruct((B, D), jnp.float32),
        scratch_types=[
            pltpu.VMEM((b_per_w,), jnp.int32),
            pltpu.VMEM((16, D), jnp.float32),
        ] + [pltpu.VMEM((C, D), jnp.float32) for _ in range(_NBUF)]
          + [pltpu.SemaphoreType.DMA for _ in range(_NBUF)],
    )
    def lookup_kernel(table_hbm, idx_hbm, out_hbm, idx_v, table_v,
                      *bufs_sems):
        rows = bufs_sems[:_NBUF]
        semo = bufs_sems[_NBUF:]
        wid = lax.axis_index("s") * NC + lax.axis_index("c")
        base = wid * b_per_w

        pltpu.sync_copy(table_hbm, table_v)
        pltpu.sync_copy(idx_hbm.at[pl.ds(base, b_per_w)], idx_v)

        def fill(g, b):
            buf = rows[b]

            def blk(jv, carry):
                vec = idx_v[pl.ds(g * C + jv * 16, 16)]
                for l in range(16):
                    r = vec[l]
                    j = jv * 16 + l
                    vals = [table_v[r, pl.ds(16 * d, 16)]
                            for d in range(n_vecs)]
                    for d in range(n_vecs):
                        buf[j, pl.ds(16 * d, 16)] = vals[d]
                return carry

            lax.fori_loop(0, C // 16, blk, 0)

        def start_out(g, b):
            pltpu.async_copy(rows[b], out_hbm.at[pl.ds(base + g * C, C)],
                             semo[b])

        def wait_out(g, b):
            pltpu.make_async_copy(rows[b],
                                  out_hbm.at[pl.ds(base + g * C, C)],
                                  semo[b]).wait()

        def body(t, carry):
            for b in range(_NBUF):
                g = _NBUF * t + b

                @pl.when(g >= _NBUF)
                def _():
                    wait_out(g, b)

                fill(g, b)
                start_out(g, b)
            return carry

        lax.fori_loop(0, n_chunks // _NBUF, body, 0)

        for b in range(_NBUF):
            wait_out(n_chunks - _NBUF + b, b)

    return lookup_kernel


def kernel(x, table):
    B = x.shape[0] * x.shape[1]
    D = table.shape[1]
    info = plsc.get_sparse_core_info()
    NC, NS = info.num_cores, info.num_subcores
    lookup = _make_lookup(B, D, NC, NS, C=64)
    out = lookup(table, x.reshape(B).astype(jnp.int32))
    return out.reshape(x.shape[0], x.shape[1], D)


# speedup vs baseline: 7.8418x; 4.7950x over previous
import functools

import jax
import jax.numpy as jnp
from jax import lax
from jax.experimental import pallas as pl
from jax.experimental.pallas import tpu as pltpu
from jax.experimental.pallas import tpu_sc as plsc

_NBUF = 2


def _make_lookup(B: int, D: int, NC: int, NS: int, C: int):
    NW = NC * NS
    b_per_w = B // NW
    n_chunks = b_per_w // C
    n_vecs = D // 16
    assert n_chunks % _NBUF == 0 and n_chunks >= 2 * _NBUF
    mesh = plsc.VectorSubcoreMesh(core_axis_name="c", subcore_axis_name="s")

    @functools.partial(
        pl.kernel,
        mesh=mesh,
        out_type=jax.ShapeDtypeStruct((B, D), jnp.float32),
        scratch_types=[
            pltpu.VMEM((b_per_w + 16,), jnp.int32),
            pltpu.VMEM((16, D), jnp.float32),
        ] + [pltpu.VMEM((C, D), jnp.float32) for _ in range(_NBUF)]
          + [pltpu.SemaphoreType.DMA for _ in range(_NBUF)],
    )
    def lookup_kernel(table_hbm, idx_hbm, out_hbm, idx_v, table_v,
                      *bufs_sems):
        rows = bufs_sems[:_NBUF]
        semo = bufs_sems[_NBUF:]
        wid = lax.axis_index("s") * NC + lax.axis_index("c")
        base = wid * b_per_w

        pltpu.sync_copy(table_hbm, table_v)
        pltpu.sync_copy(idx_hbm.at[pl.ds(base, b_per_w)],
                        idx_v.at[pl.ds(0, b_per_w)])

        def fill(g, b):
            buf = rows[b]

            @plsc.parallel_loop(0, C, unroll=2)
            def row(j):
                r = idx_v[pl.ds(g * C + j, 16)][0]
                vals = [table_v[r, pl.ds(16 * d, 16)]
                        for d in range(n_vecs)]
                for d in range(n_vecs):
                    buf[j, pl.ds(16 * d, 16)] = vals[d]

        def start_out(g, b):
            pltpu.async_copy(rows[b], out_hbm.at[pl.ds(base + g * C, C)],
                             semo[b])

        def wait_out(g, b):
            pltpu.make_async_copy(rows[b],
                                  out_hbm.at[pl.ds(base + g * C, C)],
                                  semo[b]).wait()

        def body(t, carry):
            for b in range(_NBUF):
                g = _NBUF * t + b

                @pl.when(g >= _NBUF)
                def _():
                    wait_out(g, b)

                fill(g, b)
                start_out(g, b)
            return carry

        lax.fori_loop(0, n_chunks // _NBUF, body, 0)

        for b in range(_NBUF):
            wait_out(n_chunks - _NBUF + b, b)

    return lookup_kernel


def kernel(x, table):
    B = x.shape[0] * x.shape[1]
    D = table.shape[1]
    info = plsc.get_sparse_core_info()
    NC, NS = info.num_cores, info.num_subcores
    lookup = _make_lookup(B, D, NC, NS, C=64)
    out = lookup(table, x.reshape(B).astype(jnp.int32))
    return out.reshape(x.shape[0], x.shape[1], D)
